# baseline (speedup 1.0000x reference)
import functools

import jax
import jax.numpy as jnp
import numpy as np
from jax import lax
from jax.experimental import pallas as pl
from jax.experimental.pallas import tpu as pltpu
from jax.experimental.pallas import tpu_sc as plsc

_N = 50000
_E = 800000
_EP = 819200
_IDX_ROWS = 6400
_NW = 32
_ROWS_PER_W = _IDX_ROWS // _NW
_G_CHUNK_ROWS = 4
_G_CHUNKS = _ROWS_PER_W // _G_CHUNK_ROWS
_S_CHUNK_ROWS = 8
_S_CHUNKS = _ROWS_PER_W // _S_CHUNK_ROWS
_NPAD = 50048
_NSLICE = _NPAD // 16
_EBLK = 4096
_EROWS = _EBLK // 4
_NROWS = _N // 8
_NBLKP = _NROWS

_f32 = jnp.float32


def _full_specs(arrs):
    def mk(a):
        return pl.BlockSpec(a.shape, lambda i: (0,) * a.ndim)
    return [mk(a) for a in arrs]


def _dot(x, w):
    return jnp.dot(x, w, preferred_element_type=_f32)


def _chain(x, w):
    l0, b0, l1, b1, l2, b2 = w
    h = jnp.maximum(_dot(x, l0[...]) + b0[...], 0.0)
    h = jnp.maximum(_dot(h, l1[...]) + b1[...], 0.0)
    return _dot(h, l2[...]) + b2[...]


def _edge_body(xs_ref, xn_ref, wzc_ref, wb_ref, b0_ref,
               w1_ref, b1_ref, w2_ref, b2_ref, out_ref):
    xs = xs_ref[...]
    xn = xn_ref[...]
    lane = lax.broadcasted_iota(jnp.int32, (_EROWS, 128), 1)
    z = jnp.where(lax.rem(lane, 16) < 8, xs, xs * xn)
    h = jnp.dot(z, wzc_ref[...], preferred_element_type=_f32)
    h += jnp.dot(xn, wb_ref[...], preferred_element_type=_f32)
    h = jnp.maximum(h + b0_ref[...], 0.0)
    h = jnp.maximum(
        jnp.dot(h, w1_ref[...], preferred_element_type=_f32) + b1_ref[...], 0.0)
    e = jnp.dot(h, w2_ref[...], preferred_element_type=_f32) + b2_ref[...]
    i = pl.program_id(0)
    row = lax.broadcasted_iota(jnp.int32, (_EROWS, 64), 0)
    lane64 = lax.broadcasted_iota(jnp.int32, (_EROWS, 64), 1)
    edge = (i * _EROWS + row) * 4 + lane64 // 16
    e = jnp.where(edge < _E, e, 0.0)
    half = _EROWS // 2
    out_ref[...] = jnp.concatenate([e[:half], e[half:]], axis=1)


def _tabs(x, c, perms):
    pn1, pc1, pn2, pc2 = perms
    return jnp.concatenate(
        [_dot(x, pn1[...]) + _dot(c, pc1[...]),
         _dot(x, pn2[...]) + _dot(c, pc2[...])], axis=0)


def _update_body(c_ref, p0_ref, p1_ref, *refs):
    wcat = refs[0:6]
    weat = refs[6:12]
    wapp = refs[12:20]
    wcet = refs[20:26]
    wnet = refs[26:32]
    perms = refs[32:36]
    newc_ref, tabS_ref, tabN_ref = refs[36:39]
    c = c_ref[...]
    tot = p0_ref[...] + p1_ref[...]
    ca = _chain(c, wcat)
    ea = _chain(tot, weat)
    aa, ab, ac, b0a, l1a, b1a, l2a, b2a = wapp
    h = _dot(c, aa[...]) + _dot(tot, ab[...]) + _dot(ca * ea, ac[...])
    h = jnp.maximum(h + b0a[...], 0.0)
    h = jnp.maximum(_dot(h, l1a[...]) + b1a[...], 0.0)
    nc = _dot(h, l2a[...]) + b2a[...]
    newc_ref[...] = nc
    tabS_ref[...] = _tabs(nc, _chain(nc, wcet), perms)
    tabN_ref[...] = _tabs(nc, _chain(nc, wnet), perms)


def _init_tables_body(c_ref, *refs):
    wcet = refs[0:6]
    wnet = refs[6:12]
    perms = refs[12:16]
    tabS_ref, tabN_ref = refs[16], refs[17]
    c = c_ref[...]
    tabS_ref[...] = _tabs(c, _chain(c, wcet), perms)
    tabN_ref[...] = _tabs(c, _chain(c, wnet), perms)


def _call_init_tables(cells0p, wcet, wnet, perms):
    grid = _NROWS // _NBLKP
    ws = wcet + wnet + perms
    return pl.pallas_call(
        _init_tables_body,
        grid=(grid,),
        in_specs=[pl.BlockSpec((_NBLKP, 128), lambda i: (i, 0))] + _full_specs(ws),
        out_specs=[pl.BlockSpec((2 * _NBLKP, 128), lambda i: (i, 0))] * 2,
        out_shape=[jax.ShapeDtypeStruct((2 * _NROWS, 128), _f32)] * 2,
    )(cells0p, *ws)


def _bd4(m):
    r, c = m.shape
    out = jnp.zeros((4 * r, 4 * c), _f32)
    for k in range(4):
        out = out.at[k * r:(k + 1) * r, k * c:(k + 1) * c].set(m)
    return out


def _edge_weights(params):
    w0 = params["eff_W0"]
    w1 = params["eff_W1"]
    w2 = params["eff_W2"]
    z = jnp.zeros((32, 64), _f32)
    a = z.at[0:8, 0:32].set(w0[0:8]).at[16:24, 32:64].set(w0[0:8])
    bm = z.at[0:8, 0:32].set(w0[8:16]).at[16:24, 32:64].set(w0[8:16])
    cm = z.at[8:16, 0:32].set(w0[16:24]).at[24:32, 32:64].set(w0[16:24])
    w1d = jnp.zeros((64, 64), _f32).at[0:32, 0:32].set(w1).at[32:64, 32:64].set(w1)
    w2d = jnp.zeros((64, 16), _f32).at[0:32, 0:8].set(w2).at[32:64, 8:16].set(w2)
    wzc = _bd4(a + cm)
    wb = _bd4(bm)
    b0 = jnp.tile(params["eff_b0"], 8).reshape(1, 256)
    w1q = _bd4(w1d)
    b1 = jnp.tile(params["eff_b1"], 8).reshape(1, 256)
    w2q = _bd4(w2d)
    b2 = jnp.tile(jnp.tile(params["eff_b2"], 2), 4).reshape(1, 64)
    return [wzc, wb, b0, w1q, b1, w2q, b2]


def _call_edge(xs4, xn4, weffd):
    grid = _EP // _EBLK
    data_spec = pl.BlockSpec((_EROWS, 128), lambda i: (i, 0))
    return pl.pallas_call(
        _edge_body,
        grid=(grid,),
        in_specs=[data_spec, data_spec] + _full_specs(weffd),
        out_specs=pl.BlockSpec((_EROWS // 2, 128), lambda i: (i, 0)),
        out_shape=jax.ShapeDtypeStruct((_EP // 8, 128), _f32),
    )(xs4, xn4, *weffd)


def _call_update(cellsp, p0p, p1p, wcat, weat, wapp, wcet, wnet, perms):
    grid = _NROWS // _NBLKP
    d128 = pl.BlockSpec((_NBLKP, 128), lambda i: (i, 0))
    ws = wcat + weat + wapp + wcet + wnet + perms
    return pl.pallas_call(
        _update_body,
        grid=(grid,),
        in_specs=[d128, d128, d128] + _full_specs(ws),
        out_specs=[d128,
                   pl.BlockSpec((2 * _NBLKP, 128), lambda i: (i, 0)),
                   pl.BlockSpec((2 * _NBLKP, 128), lambda i: (i, 0))],
        out_shape=[
            jax.ShapeDtypeStruct((_NROWS, 128), _f32),
            jax.ShapeDtypeStruct((2 * _NROWS, 128), _f32),
            jax.ShapeDtypeStruct((2 * _NROWS, 128), _f32),
        ],
    )(cellsp, p0p, p1p, *ws)


def _embed16(w):
    z = jnp.zeros((16, 64), _f32)
    return z.at[0:8, 0:32].set(w).at[8:16, 32:64].set(w)


def _bd(m, reps):
    r, c = m.shape
    out = jnp.zeros((reps * r, reps * c), _f32)
    for k in range(reps):
        out = out.at[k * r:(k + 1) * r, k * c:(k + 1) * c].set(m)
    return out


def _w2d16(w):
    return jnp.zeros((64, 16), _f32).at[0:32, 0:8].set(w).at[32:64, 8:16].set(w)


def _node_weights(params, prefix):
    w0 = params[prefix + "_W0"]
    l0 = _bd(_embed16(w0), 8)
    l1 = _bd(_bd(params[prefix + "_W1"], 2), 8)
    l2 = _bd(_w2d16(params[prefix + "_W2"]), 8)
    b0 = jnp.tile(params[prefix + "_b0"], 16).reshape(1, 512)
    b1 = jnp.tile(params[prefix + "_b1"], 16).reshape(1, 512)
    b2 = jnp.tile(params[prefix + "_b2"], 16).reshape(1, 128)
    return [l0, b0, l1, b1, l2, b2]


def _app_weights(params):
    w0 = params["app_W0"]
    aa = _bd(_embed16(w0[0:8]), 8)
    ab = _bd(_embed16(w0[8:16]), 8)
    ac = _bd(_embed16(w0[16:24]), 8)
    l1 = _bd(_bd(params["app_W1"], 2), 8)
    l2 = _bd(_w2d16(params["app_W2"]), 8)
    b0 = jnp.tile(params["app_b0"], 16).reshape(1, 512)
    b1 = jnp.tile(params["app_b1"], 16).reshape(1, 512)
    b2 = jnp.tile(params["app_b2"], 16).reshape(1, 128)
    return [aa, ab, ac, b0, l1, b1, l2, b2]


def _perm_mats():
    pn1 = np.zeros((128, 128), np.float32)
    pc1 = np.zeros((128, 128), np.float32)
    pn2 = np.zeros((128, 128), np.float32)
    pc2 = np.zeros((128, 128), np.float32)
    eye = np.eye(8, dtype=np.float32)
    for k in range(4):
        pn1[16 * k:16 * k + 8, 32 * k:32 * k + 8] = eye
        pn1[16 * k + 8:16 * k + 16, 32 * k + 16:32 * k + 24] = eye
        pc1[16 * k:16 * k + 8, 32 * k + 8:32 * k + 16] = eye
        pc1[16 * k + 8:16 * k + 16, 32 * k + 24:32 * k + 32] = eye
        pn2[64 + 16 * k:64 + 16 * k + 8, 32 * k:32 * k + 8] = eye
        pn2[64 + 16 * k + 8:64 + 16 * k + 16, 32 * k + 16:32 * k + 24] = eye
        pc2[64 + 16 * k:64 + 16 * k + 8, 32 * k + 8:32 * k + 16] = eye
        pc2[64 + 16 * k + 8:64 + 16 * k + 16, 32 * k + 24:32 * k + 32] = eye
    return [jnp.asarray(pn1), jnp.asarray(pc1), jnp.asarray(pn2), jnp.asarray(pc2)]


def _tab_row_of_node(n):
    blk = n // (8 * _NBLKP)
    j = n % (8 * _NBLKP)
    return (blk * (8 * _NBLKP) + ((j % 8) // 4) * (4 * _NBLKP)
            + (j // 8) * 4 + (j % 4))


@functools.cache
def _sc_kernels():
    mesh = plsc.VectorSubcoreMesh(core_axis_name="c", subcore_axis_name="s")
    gather = functools.partial(
        pl.kernel,
        out_type=[
            jax.ShapeDtypeStruct((_EP, 32), _f32),
            jax.ShapeDtypeStruct((_EP, 32), _f32),
        ],
        mesh=mesh,
        scratch_types=[
            pltpu.VMEM((2, _G_CHUNK_ROWS, 128), jnp.int32),
            pltpu.VMEM((2, _G_CHUNK_ROWS, 128), jnp.int32),
            pltpu.VMEM((2, _G_CHUNK_ROWS * 128, 32), _f32),
            pltpu.VMEM((2, _G_CHUNK_ROWS * 128, 32), _f32),
            pltpu.SemaphoreType.DMA,
            pltpu.SemaphoreType.DMA,
            pltpu.SemaphoreType.DMA,
            pltpu.SemaphoreType.DMA,
        ],
        compiler_params=pltpu.CompilerParams(use_tc_tiling_on_sc=False),
    )
    scatter = functools.partial(
        pl.kernel,
        out_type=jax.ShapeDtypeStruct((2 * _NPAD, 16), _f32),
        mesh=mesh,
        scratch_types=[
            pltpu.VMEM((_S_CHUNK_ROWS, 128), jnp.int32),
            pltpu.VMEM((_S_CHUNK_ROWS * 128, 16), _f32),
            pltpu.VMEM_SHARED((_NPAD, 16), _f32),
        ],
        compiler_params=pltpu.CompilerParams(use_tc_tiling_on_sc=False),
    )
    return gather(_sc_gather_body), scatter(_sc_scatter_body)


def _sc_gather_body(tabS, tabN, seg2d, nbr2d, xs_out, xn_out,
                    segv, nbrv, bufS, bufN, semS0, semN0, semS1, semN1):
    wid = lax.axis_index("s") * 2 + lax.axis_index("c")
    sems = ((semS0, semN0), (semS1, semN1))

    def fire(k, par):
        semS, semN = sems[par]
        row0 = wid * _ROWS_PER_W + k * _G_CHUNK_ROWS
        pltpu.sync_copy(seg2d.at[pl.ds(row0, _G_CHUNK_ROWS)], segv.at[par])
        pltpu.sync_copy(nbr2d.at[pl.ds(row0, _G_CHUNK_ROWS)], nbrv.at[par])

        def sub(j, c2):
            pltpu.async_copy(tabS.at[segv.at[par].at[j]],
                             bufS.at[par].at[pl.ds(j * 128, 128)], semS)
            pltpu.async_copy(tabN.at[nbrv.at[par].at[j]],
                             bufN.at[par].at[pl.ds(j * 128, 128)], semN)
            return c2

        lax.fori_loop(0, _G_CHUNK_ROWS, sub, 0)

    def drain_write(k, par):
        semS, semN = sems[par]
        pltpu.make_async_copy(tabS.at[pl.ds(0, _G_CHUNK_ROWS * 128)],
                              bufS.at[par], semS).wait()
        pltpu.make_async_copy(tabN.at[pl.ds(0, _G_CHUNK_ROWS * 128)],
                              bufN.at[par], semN).wait()
        e0 = (wid * _ROWS_PER_W + k * _G_CHUNK_ROWS) * 128
        pltpu.sync_copy(bufS.at[par], xs_out.at[pl.ds(e0, _G_CHUNK_ROWS * 128)])
        pltpu.sync_copy(bufN.at[par], xn_out.at[pl.ds(e0, _G_CHUNK_ROWS * 128)])

    fire(0, 0)

    def pair(i, carry):
        k0 = 2 * i

        fire(k0 + 1, 1)
        drain_write(k0, 0)

        @pl.when(k0 + 2 < _G_CHUNKS)
        def _():
            fire(k0 + 2, 0)

        drain_write(k0 + 1, 1)
        return carry

    lax.fori_loop(0, _G_CHUNKS // 2, pair, 0)


def _sc_scatter_body(eff, seg2d, zeros_tab, out, segv, valv, shared):
    cid = lax.axis_index("c")
    sid = lax.axis_index("s")
    wid = sid * 2 + cid

    pltpu.sync_copy(zeros_tab.at[pl.ds(sid * _NSLICE, _NSLICE)],
                    shared.at[pl.ds(sid * _NSLICE, _NSLICE)])
    plsc.subcore_barrier()

    def chunk(k, carry):
        row0 = wid * _ROWS_PER_W + k * _S_CHUNK_ROWS
        pltpu.sync_copy(seg2d.at[pl.ds(row0, _S_CHUNK_ROWS)], segv)
        pltpu.sync_copy(eff.at[pl.ds(row0 * 128, _S_CHUNK_ROWS * 128)], valv)

        def sub(j, c2):
            pltpu.sync_copy(valv.at[pl.ds(j * 128, 128)],
                            shared.at[segv.at[j]], add=True)
            return c2

        lax.fori_loop(0, _S_CHUNK_ROWS, sub, 0)
        return carry

    lax.fori_loop(0, _S_CHUNKS, chunk, 0)
    plsc.subcore_barrier()
    pltpu.sync_copy(shared.at[pl.ds(sid * _NSLICE, _NSLICE)],
                    out.at[pl.ds(cid * _NPAD + sid * _NSLICE, _NSLICE)])


def kernel(grid_obs, edge_index, params):
    seg = edge_index[0]
    nbr = edge_index[1]
    pad = jnp.zeros((_EP - _E,), jnp.int32)
    seg_pad = jnp.concatenate([seg, pad])
    nbr_pad = jnp.concatenate([nbr, pad])
    seg2d = _tab_row_of_node(seg_pad).reshape(_IDX_ROWS, 128)
    nbr2d = _tab_row_of_node(nbr_pad).reshape(_IDX_ROWS, 128)

    seg2d_scat = (seg_pad.reshape(_EP // _EBLK, 2, _EBLK // 8, 4)
                  .transpose(0, 2, 1, 3).reshape(_IDX_ROWS, 128))

    zeros_hid = jnp.zeros((_N, 4), _f32)
    cells16 = jnp.concatenate(
        [grid_obs[0], zeros_hid, grid_obs[1], zeros_hid], axis=1)
    cellsp = cells16.reshape(_NROWS, 128)
    zeros_tab = jnp.zeros((_NPAD, 16), _f32)

    wcet = _node_weights(params, "cet")
    wnet = _node_weights(params, "net")
    wcat = _node_weights(params, "cat")
    weat = _node_weights(params, "eat")
    wapp = _app_weights(params)
    weffd = _edge_weights(params)
    perms = _perm_mats()

    tabS, tabN = _call_init_tables(cellsp, wcet, wnet, perms)
    sc_gather, sc_scatter = _sc_kernels()

    preds = []
    for _ in range(2):
        xs, xn = sc_gather(tabS.reshape(_N, 32), tabN.reshape(_N, 32),
                           seg2d, nbr2d)
        eff8 = _call_edge(xs.reshape(_EP // 4, 128), xn.reshape(_EP // 4, 128),
                          weffd)
        partials = sc_scatter(eff8.reshape(_EP, 16), seg2d_scat, zeros_tab)
        cellsp, tabS, tabN = _call_update(
            cellsp,
            partials[:_N].reshape(_NROWS, 128),
            partials[_NPAD:_NPAD + _N].reshape(_NROWS, 128),
            wcat, weat, wapp, wcet, wnet, perms)
        nc16 = cellsp.reshape(_N, 16)
        preds.append(jnp.stack([nc16[:, 0:4], nc16[:, 8:12]], axis=0))

    return jnp.stack(preds, axis=1)


# speedup vs baseline: 1.0210x; 1.0210x over previous
import functools

import jax
import jax.numpy as jnp
import numpy as np
from jax import lax
from jax.experimental import pallas as pl
from jax.experimental.pallas import tpu as pltpu
from jax.experimental.pallas import tpu_sc as plsc

_N = 50000
_E = 800000
_EP = 819200
_IDX_ROWS = 6400
_NW = 32
_ROWS_PER_W = _IDX_ROWS // _NW
_G_CHUNK_ROWS = 4
_G_CHUNKS = _ROWS_PER_W // _G_CHUNK_ROWS
_S_CHUNK_ROWS = 8
_S_CHUNKS = _ROWS_PER_W // _S_CHUNK_ROWS
_NPAD = 50048
_NSLICE = _NPAD // 16
_EBLK = 4096
_EROWS = _EBLK // 4
_NBLK = 2000

_f32 = jnp.float32


def _full_specs(arrs):
    def mk(a):
        return pl.BlockSpec(a.shape, lambda i: (0,) * a.ndim)
    return [mk(a) for a in arrs]


def _dot(x, w):
    return jnp.dot(x, w, preferred_element_type=_f32)


def _chain(x, w):
    l0, b0, l1, b1, l2, b2 = w
    h = jnp.maximum(_dot(x, l0[...]) + b0[...], 0.0)
    h = jnp.maximum(_dot(h, l1[...]) + b1[...], 0.0)
    return _dot(h, l2[...]) + b2[...]


def _edge_body(xs_ref, xn_ref, wzc_ref, wb_ref, b0_ref,
               w1_ref, b1_ref, w2_ref, b2_ref, out_ref):
    xs = xs_ref[...]
    xn = xn_ref[...]
    lane = lax.broadcasted_iota(jnp.int32, (_EROWS, 128), 1)
    z = jnp.where(lax.rem(lane, 16) < 8, xs, xs * xn)
    h = jnp.dot(z, wzc_ref[...], preferred_element_type=_f32)
    h += jnp.dot(xn, wb_ref[...], preferred_element_type=_f32)
    h = jnp.maximum(h + b0_ref[...], 0.0)
    h = jnp.maximum(
        jnp.dot(h, w1_ref[...], preferred_element_type=_f32) + b1_ref[...], 0.0)
    e = jnp.dot(h, w2_ref[...], preferred_element_type=_f32) + b2_ref[...]
    i = pl.program_id(0)
    row = lax.broadcasted_iota(jnp.int32, (_EROWS, 64), 0)
    lane64 = lax.broadcasted_iota(jnp.int32, (_EROWS, 64), 1)
    edge = (i * _EROWS + row) * 4 + lane64 // 16
    e = jnp.where(edge < _E, e, 0.0)
    half = _EROWS // 2
    out_ref[...] = jnp.concatenate([e[:half], e[half:]], axis=1)


def _wlist(params, prefix):
    out = []
    for i in range(3):
        out.append(params[prefix + "_W" + str(i)])
        out.append(params[prefix + "_b" + str(i)].reshape(1, -1))
    return out


def _mlp3(x, refs):
    w0, b0, w1, b1, w2, b2 = refs
    h = jnp.maximum(_dot(x, w0[...]) + b0[...], 0.0)
    h = jnp.maximum(_dot(h, w1[...]) + b1[...], 0.0)
    return _dot(h, w2[...]) + b2[...]


def _update_body(c_ref, p0_ref, p1_ref, *refs):
    wcat = refs[0:6]
    weat = refs[6:12]
    wapp = refs[12:18]
    wcet = refs[18:24]
    wnet = refs[24:30]
    pred_ref, newc_ref, tabS_ref, tabN_ref = refs[30:34]
    c = c_ref[...]
    tot = p0_ref[...] + p1_ref[...]
    n = c.shape[0]
    cs = jnp.concatenate([c[:, :8], c[:, 8:]], axis=0)
    ts = jnp.concatenate([tot[:, :8], tot[:, 8:]], axis=0)
    ca = _mlp3(cs, wcat)
    ea = _mlp3(ts, weat)
    ain = jnp.concatenate([cs, ts, ca * ea], axis=1)
    nc = _mlp3(ain, wapp)
    pred_ref[...] = jnp.concatenate([nc[:n, :4], nc[n:, :4]], axis=1)
    newc_ref[...] = jnp.concatenate([nc[:n], nc[n:]], axis=1)
    C = _mlp3(nc, wcet)
    D = _mlp3(nc, wnet)
    tabS_ref[...] = jnp.concatenate([nc[:n], C[:n], nc[n:], C[n:]], axis=1)
    tabN_ref[...] = jnp.concatenate([nc[:n], D[:n], nc[n:], D[n:]], axis=1)


def _init_tables_body(c_ref, *refs):
    wcet = refs[0:6]
    wnet = refs[6:12]
    tabS_ref, tabN_ref = refs[12], refs[13]
    c = c_ref[...]
    cs = jnp.concatenate([c[:, :8], c[:, 8:]], axis=0)
    C = _mlp3(cs, wcet)
    D = _mlp3(cs, wnet)
    n = c.shape[0]
    tabS_ref[...] = jnp.concatenate([c[:, :8], C[:n], c[:, 8:], C[n:]], axis=1)
    tabN_ref[...] = jnp.concatenate([c[:, :8], D[:n], c[:, 8:], D[n:]], axis=1)


def _call_init_tables(cells0, wcet, wnet):
    grid = _N // _NBLK
    data_spec = pl.BlockSpec((_NBLK, 16), lambda i: (i, 0))
    out_spec = pl.BlockSpec((_NBLK, 32), lambda i: (i, 0))
    return pl.pallas_call(
        _init_tables_body,
        grid=(grid,),
        in_specs=[data_spec] + _full_specs(wcet) + _full_specs(wnet),
        out_specs=[out_spec, out_spec],
        out_shape=[jax.ShapeDtypeStruct((_N, 32), _f32)] * 2,
    )(cells0, *wcet, *wnet)


def _bd4(m):
    r, c = m.shape
    out = jnp.zeros((4 * r, 4 * c), _f32)
    for k in range(4):
        out = out.at[k * r:(k + 1) * r, k * c:(k + 1) * c].set(m)
    return out


def _edge_weights(params):
    w0 = params["eff_W0"]
    w1 = params["eff_W1"]
    w2 = params["eff_W2"]
    z = jnp.zeros((32, 64), _f32)
    a = z.at[0:8, 0:32].set(w0[0:8]).at[16:24, 32:64].set(w0[0:8])
    bm = z.at[0:8, 0:32].set(w0[8:16]).at[16:24, 32:64].set(w0[8:16])
    cm = z.at[8:16, 0:32].set(w0[16:24]).at[24:32, 32:64].set(w0[16:24])
    w1d = jnp.zeros((64, 64), _f32).at[0:32, 0:32].set(w1).at[32:64, 32:64].set(w1)
    w2d = jnp.zeros((64, 16), _f32).at[0:32, 0:8].set(w2).at[32:64, 8:16].set(w2)
    wzc = _bd4(a + cm)
    wb = _bd4(bm)
    b0 = jnp.tile(params["eff_b0"], 8).reshape(1, 256)
    w1q = _bd4(w1d)
    b1 = jnp.tile(params["eff_b1"], 8).reshape(1, 256)
    w2q = _bd4(w2d)
    b2 = jnp.tile(jnp.tile(params["eff_b2"], 2), 4).reshape(1, 64)
    return [wzc, wb, b0, w1q, b1, w2q, b2]


def _call_edge(xs4, xn4, weffd):
    grid = _EP // _EBLK
    data_spec = pl.BlockSpec((_EROWS, 128), lambda i: (i, 0))
    return pl.pallas_call(
        _edge_body,
        grid=(grid,),
        in_specs=[data_spec, data_spec] + _full_specs(weffd),
        out_specs=pl.BlockSpec((_EROWS // 2, 128), lambda i: (i, 0)),
        out_shape=jax.ShapeDtypeStruct((_EP // 8, 128), _f32),
    )(xs4, xn4, *weffd)


def _call_update(cells, p0, p1, wcat, weat, wapp, wcet, wnet):
    grid = _N // _NBLK
    d16 = pl.BlockSpec((_NBLK, 16), lambda i: (i, 0))
    d8 = pl.BlockSpec((_NBLK, 8), lambda i: (i, 0))
    d32 = pl.BlockSpec((_NBLK, 32), lambda i: (i, 0))
    ws = wcat + weat + wapp + wcet + wnet
    return pl.pallas_call(
        _update_body,
        grid=(grid,),
        in_specs=[d16, d16, d16] + _full_specs(ws),
        out_specs=[d8, d16, d32, d32],
        out_shape=[
            jax.ShapeDtypeStruct((_N, 8), _f32),
            jax.ShapeDtypeStruct((_N, 16), _f32),
            jax.ShapeDtypeStruct((_N, 32), _f32),
            jax.ShapeDtypeStruct((_N, 32), _f32),
        ],
    )(cells, p0, p1, *ws)


@functools.cache
def _sc_kernels():
    mesh = plsc.VectorSubcoreMesh(core_axis_name="c", subcore_axis_name="s")
    gather = functools.partial(
        pl.kernel,
        out_type=[
            jax.ShapeDtypeStruct((_EP, 32), _f32),
            jax.ShapeDtypeStruct((_EP, 32), _f32),
        ],
        mesh=mesh,
        scratch_types=[
            pltpu.VMEM((2, _G_CHUNK_ROWS, 128), jnp.int32),
            pltpu.VMEM((2, _G_CHUNK_ROWS, 128), jnp.int32),
            pltpu.VMEM((2, _G_CHUNK_ROWS * 128, 32), _f32),
            pltpu.VMEM((2, _G_CHUNK_ROWS * 128, 32), _f32),
            pltpu.SemaphoreType.DMA,
            pltpu.SemaphoreType.DMA,
            pltpu.SemaphoreType.DMA,
            pltpu.SemaphoreType.DMA,
        ],
        compiler_params=pltpu.CompilerParams(use_tc_tiling_on_sc=False),
    )
    scatter = functools.partial(
        pl.kernel,
        out_type=jax.ShapeDtypeStruct((2 * _NPAD, 16), _f32),
        mesh=mesh,
        scratch_types=[
            pltpu.VMEM((_S_CHUNK_ROWS, 128), jnp.int32),
            pltpu.VMEM((_S_CHUNK_ROWS * 128, 16), _f32),
            pltpu.VMEM_SHARED((_NPAD, 16), _f32),
        ],
        compiler_params=pltpu.CompilerParams(use_tc_tiling_on_sc=False),
    )
    return gather(_sc_gather_body), scatter(_sc_scatter_body)


def _sc_gather_body(tabS, tabN, seg2d, nbr2d, xs_out, xn_out,
                    segv, nbrv, bufS, bufN, semS0, semN0, semS1, semN1):
    wid = lax.axis_index("s") * 2 + lax.axis_index("c")
    sems = ((semS0, semN0), (semS1, semN1))

    def fire(k, par):
        semS, semN = sems[par]
        row0 = wid * _ROWS_PER_W + k * _G_CHUNK_ROWS
        pltpu.sync_copy(seg2d.at[pl.ds(row0, _G_CHUNK_ROWS)], segv.at[par])
        pltpu.sync_copy(nbr2d.at[pl.ds(row0, _G_CHUNK_ROWS)], nbrv.at[par])

        def sub(j, c2):
            pltpu.async_copy(tabS.at[segv.at[par].at[j]],
                             bufS.at[par].at[pl.ds(j * 128, 128)], semS)
            pltpu.async_copy(tabN.at[nbrv.at[par].at[j]],
                             bufN.at[par].at[pl.ds(j * 128, 128)], semN)
            return c2

        lax.fori_loop(0, _G_CHUNK_ROWS, sub, 0)

    def drain_write(k, par):
        semS, semN = sems[par]
        pltpu.make_async_copy(tabS.at[pl.ds(0, _G_CHUNK_ROWS * 128)],
                              bufS.at[par], semS).wait()
        pltpu.make_async_copy(tabN.at[pl.ds(0, _G_CHUNK_ROWS * 128)],
                              bufN.at[par], semN).wait()
        e0 = (wid * _ROWS_PER_W + k * _G_CHUNK_ROWS) * 128
        pltpu.sync_copy(bufS.at[par], xs_out.at[pl.ds(e0, _G_CHUNK_ROWS * 128)])
        pltpu.sync_copy(bufN.at[par], xn_out.at[pl.ds(e0, _G_CHUNK_ROWS * 128)])

    fire(0, 0)

    def pair(i, carry):
        k0 = 2 * i

        fire(k0 + 1, 1)
        drain_write(k0, 0)

        @pl.when(k0 + 2 < _G_CHUNKS)
        def _():
            fire(k0 + 2, 0)

        drain_write(k0 + 1, 1)
        return carry

    lax.fori_loop(0, _G_CHUNKS // 2, pair, 0)


def _sc_scatter_body(eff, seg2d, zeros_tab, out, segv, valv, shared):
    cid = lax.axis_index("c")
    sid = lax.axis_index("s")
    wid = sid * 2 + cid

    pltpu.sync_copy(zeros_tab.at[pl.ds(sid * _NSLICE, _NSLICE)],
                    shared.at[pl.ds(sid * _NSLICE, _NSLICE)])
    plsc.subcore_barrier()

    def chunk(k, carry):
        row0 = wid * _ROWS_PER_W + k * _S_CHUNK_ROWS
        pltpu.sync_copy(seg2d.at[pl.ds(row0, _S_CHUNK_ROWS)], segv)
        pltpu.sync_copy(eff.at[pl.ds(row0 * 128, _S_CHUNK_ROWS * 128)], valv)

        def sub(j, c2):
            pltpu.sync_copy(valv.at[pl.ds(j * 128, 128)],
                            shared.at[segv.at[j]], add=True)
            return c2

        lax.fori_loop(0, _S_CHUNK_ROWS, sub, 0)
        return carry

    lax.fori_loop(0, _S_CHUNKS, chunk, 0)
    plsc.subcore_barrier()
    pltpu.sync_copy(shared.at[pl.ds(sid * _NSLICE, _NSLICE)],
                    out.at[pl.ds(cid * _NPAD + sid * _NSLICE, _NSLICE)])


def kernel(grid_obs, edge_index, params):
    seg = edge_index[0]
    nbr = edge_index[1]
    pad = jnp.zeros((_EP - _E,), jnp.int32)
    seg_pad = jnp.concatenate([seg, pad])
    seg2d = seg_pad.reshape(_IDX_ROWS, 128)
    nbr2d = jnp.concatenate([nbr, pad]).reshape(_IDX_ROWS, 128)

    seg2d_scat = (seg_pad.reshape(_EP // _EBLK, 2, _EBLK // 8, 4)
                  .transpose(0, 2, 1, 3).reshape(_IDX_ROWS, 128))

    zeros_hid = jnp.zeros((_N, 4), _f32)
    cells = jnp.concatenate(
        [grid_obs[0], zeros_hid, grid_obs[1], zeros_hid], axis=1)
    zeros_tab = jnp.zeros((_NPAD, 16), _f32)

    wcet = _wlist(params, "cet")
    wnet = _wlist(params, "net")
    wcat = _wlist(params, "cat")
    weat = _wlist(params, "eat")
    wapp = _wlist(params, "app")
    weffd = _edge_weights(params)

    tabS, tabN = _call_init_tables(cells, wcet, wnet)
    sc_gather, sc_scatter = _sc_kernels()

    preds = []
    for _ in range(2):
        xs, xn = sc_gather(tabS, tabN, seg2d, nbr2d)
        eff8 = _call_edge(xs.reshape(_EP // 4, 128), xn.reshape(_EP // 4, 128),
                          weffd)
        partials = sc_scatter(eff8.reshape(_EP, 16), seg2d_scat, zeros_tab)
        pred, cells, tabS, tabN = _call_update(
            cells, partials[:_N], partials[_NPAD:_NPAD + _N],
            wcat, weat, wapp, wcet, wnet)
        preds.append(pred.reshape(_N, 2, 4).transpose(1, 0, 2))

    return jnp.stack(preds, axis=1)


# speedup vs baseline: 1.1071x; 1.0843x over previous
import functools

import jax
import jax.numpy as jnp
import numpy as np
from jax import lax
from jax.experimental import pallas as pl
from jax.experimental.pallas import tpu as pltpu
from jax.experimental.pallas import tpu_sc as plsc

_N = 50000
_E = 800000
_EP = 819200
_IDX_ROWS = 6400
_NW = 32
_ROWS_PER_W = _IDX_ROWS // _NW
_G_CHUNK_ROWS = 4
_G_CHUNKS = _ROWS_PER_W // _G_CHUNK_ROWS
_S_CHUNK_ROWS = 8
_S_CHUNKS = _ROWS_PER_W // _S_CHUNK_ROWS
_NPAD = 50048
_NSLICE = _NPAD // 16
_EBLK = 4096
_EROWS = _EBLK // 4
_NBLK = 2000

_f32 = jnp.float32


def _full_specs(arrs):
    def mk(a):
        return pl.BlockSpec(a.shape, lambda i: (0,) * a.ndim)
    return [mk(a) for a in arrs]


def _dot(x, w):
    return jnp.dot(x, w, preferred_element_type=_f32)


def _chain(x, w):
    l0, b0, l1, b1, l2, b2 = w
    h = jnp.maximum(_dot(x, l0[...]) + b0[...], 0.0)
    h = jnp.maximum(_dot(h, l1[...]) + b1[...], 0.0)
    return _dot(h, l2[...]) + b2[...]


def _edge_body(xs_ref, xn_ref, wzc_ref, wb_ref, b0_ref,
               w1_ref, b1_ref, w2_ref, b2_ref, out_ref):
    xs = xs_ref[...]
    xn = xn_ref[...]
    lane = lax.broadcasted_iota(jnp.int32, (_EROWS, 128), 1)
    z = jnp.where(lax.rem(lane, 16) < 8, xs, xs * xn)
    h = jnp.dot(z, wzc_ref[...], preferred_element_type=_f32)
    h += jnp.dot(xn, wb_ref[...], preferred_element_type=_f32)
    h = jnp.maximum(h + b0_ref[...], 0.0)
    h = jnp.maximum(
        jnp.dot(h, w1_ref[...], preferred_element_type=_f32) + b1_ref[...], 0.0)
    e = jnp.dot(h, w2_ref[...], preferred_element_type=_f32) + b2_ref[...]
    i = pl.program_id(0)
    row = lax.broadcasted_iota(jnp.int32, (_EROWS, 64), 0)
    lane64 = lax.broadcasted_iota(jnp.int32, (_EROWS, 64), 1)
    edge = (i * _EROWS + row) * 4 + lane64 // 16
    e = jnp.where(edge < _E, e, 0.0)
    half = _EROWS // 2
    out_ref[...] = jnp.concatenate([e[:half], e[half:]], axis=1)


def _wlist(params, prefix):
    out = []
    for i in range(3):
        out.append(params[prefix + "_W" + str(i)])
        out.append(params[prefix + "_b" + str(i)].reshape(1, -1))
    return out


def _mlp3(x, refs):
    w0, b0, w1, b1, w2, b2 = refs
    h = jnp.maximum(_dot(x, w0[...]) + b0[...], 0.0)
    h = jnp.maximum(_dot(h, w1[...]) + b1[...], 0.0)
    return _dot(h, w2[...]) + b2[...]


def _update_body(c_ref, p0_ref, p1_ref, *refs):
    wcat = refs[0:6]
    weat = refs[6:12]
    wapp = refs[12:18]
    wcet = refs[18:24]
    wnet = refs[24:30]
    pred_ref, newc_ref, tabS_ref, tabN_ref = refs[30:34]
    c = c_ref[...]
    tot = p0_ref[...] + p1_ref[...]
    n = c.shape[0]
    cs = jnp.concatenate([c[:, :8], c[:, 8:]], axis=0)
    ts = jnp.concatenate([tot[:, :8], tot[:, 8:]], axis=0)
    ca = _mlp3(cs, wcat)
    ea = _mlp3(ts, weat)
    ain = jnp.concatenate([cs, ts, ca * ea], axis=1)
    nc = _mlp3(ain, wapp)
    pred_ref[...] = jnp.concatenate([nc[:n, :4], nc[n:, :4]], axis=1)
    newc_ref[...] = jnp.concatenate([nc[:n], nc[n:]], axis=1)
    C = _mlp3(nc, wcet)
    D = _mlp3(nc, wnet)
    tabS_ref[...] = jnp.concatenate([nc[:n], C[:n], nc[n:], C[n:]], axis=1)
    tabN_ref[...] = jnp.concatenate([nc[:n], D[:n], nc[n:], D[n:]], axis=1)


def _init_tables_body(c_ref, *refs):
    wcet = refs[0:6]
    wnet = refs[6:12]
    tabS_ref, tabN_ref = refs[12], refs[13]
    c = c_ref[...]
    cs = jnp.concatenate([c[:, :8], c[:, 8:]], axis=0)
    C = _mlp3(cs, wcet)
    D = _mlp3(cs, wnet)
    n = c.shape[0]
    tabS_ref[...] = jnp.concatenate([c[:, :8], C[:n], c[:, 8:], C[n:]], axis=1)
    tabN_ref[...] = jnp.concatenate([c[:, :8], D[:n], c[:, 8:], D[n:]], axis=1)


def _call_init_tables(cells0, wcet, wnet):
    grid = _N // _NBLK
    data_spec = pl.BlockSpec((_NBLK, 16), lambda i: (i, 0))
    out_spec = pl.BlockSpec((_NBLK, 32), lambda i: (i, 0))
    return pl.pallas_call(
        _init_tables_body,
        grid=(grid,),
        in_specs=[data_spec] + _full_specs(wcet) + _full_specs(wnet),
        out_specs=[out_spec, out_spec],
        out_shape=[jax.ShapeDtypeStruct((_N, 32), _f32)] * 2,
    )(cells0, *wcet, *wnet)


def _bd4(m):
    r, c = m.shape
    out = jnp.zeros((4 * r, 4 * c), _f32)
    for k in range(4):
        out = out.at[k * r:(k + 1) * r, k * c:(k + 1) * c].set(m)
    return out


def _edge_weights(params):
    w0 = params["eff_W0"]
    w1 = params["eff_W1"]
    w2 = params["eff_W2"]
    z = jnp.zeros((32, 64), _f32)
    a = z.at[0:8, 0:32].set(w0[0:8]).at[16:24, 32:64].set(w0[0:8])
    bm = z.at[0:8, 0:32].set(w0[8:16]).at[16:24, 32:64].set(w0[8:16])
    cm = z.at[8:16, 0:32].set(w0[16:24]).at[24:32, 32:64].set(w0[16:24])
    w1d = jnp.zeros((64, 64), _f32).at[0:32, 0:32].set(w1).at[32:64, 32:64].set(w1)
    w2d = jnp.zeros((64, 16), _f32).at[0:32, 0:8].set(w2).at[32:64, 8:16].set(w2)
    wzc = _bd4(a + cm)
    wb = _bd4(bm)
    b0 = jnp.tile(params["eff_b0"], 8).reshape(1, 256)
    w1q = _bd4(w1d)
    b1 = jnp.tile(params["eff_b1"], 8).reshape(1, 256)
    w2q = _bd4(w2d)
    b2 = jnp.tile(jnp.tile(params["eff_b2"], 2), 4).reshape(1, 64)
    return [wzc, wb, b0, w1q, b1, w2q, b2]


def _call_edge(xs4, xn4, weffd):
    grid = _EP // _EBLK
    data_spec = pl.BlockSpec((_EROWS, 128), lambda i: (i, 0))
    return pl.pallas_call(
        _edge_body,
        grid=(grid,),
        in_specs=[data_spec, data_spec] + _full_specs(weffd),
        out_specs=pl.BlockSpec((_EROWS // 2, 128), lambda i: (i, 0)),
        out_shape=jax.ShapeDtypeStruct((_EP // 8, 128), _f32),
    )(xs4, xn4, *weffd)


def _call_update(cells, p0, p1, wcat, weat, wapp, wcet, wnet):
    grid = _N // _NBLK
    d16 = pl.BlockSpec((_NBLK, 16), lambda i: (i, 0))
    d8 = pl.BlockSpec((_NBLK, 8), lambda i: (i, 0))
    d32 = pl.BlockSpec((_NBLK, 32), lambda i: (i, 0))
    ws = wcat + weat + wapp + wcet + wnet
    return pl.pallas_call(
        _update_body,
        grid=(grid,),
        in_specs=[d16, d16, d16] + _full_specs(ws),
        out_specs=[d8, d16, d32, d32],
        out_shape=[
            jax.ShapeDtypeStruct((_N, 8), _f32),
            jax.ShapeDtypeStruct((_N, 16), _f32),
            jax.ShapeDtypeStruct((_N, 32), _f32),
            jax.ShapeDtypeStruct((_N, 32), _f32),
        ],
    )(cells, p0, p1, *ws)


@functools.cache
def _sc_kernels():
    mesh = plsc.VectorSubcoreMesh(core_axis_name="c", subcore_axis_name="s")
    gather = functools.partial(
        pl.kernel,
        out_type=[
            jax.ShapeDtypeStruct((_EP, 32), _f32),
            jax.ShapeDtypeStruct((_EP, 32), _f32),
        ],
        mesh=mesh,
        scratch_types=[
            pltpu.VMEM((2, _G_CHUNK_ROWS, 128), jnp.int32),
            pltpu.VMEM((2, _G_CHUNK_ROWS, 128), jnp.int32),
            pltpu.VMEM((2, _G_CHUNK_ROWS * 128, 32), _f32),
            pltpu.VMEM((2, _G_CHUNK_ROWS * 128, 32), _f32),
            pltpu.SemaphoreType.DMA,
            pltpu.SemaphoreType.DMA,
            pltpu.SemaphoreType.DMA,
            pltpu.SemaphoreType.DMA,
        ],
        compiler_params=pltpu.CompilerParams(use_tc_tiling_on_sc=False),
    )
    scatter = functools.partial(
        pl.kernel,
        out_type=jax.ShapeDtypeStruct((2 * _NPAD, 16), _f32),
        mesh=mesh,
        scratch_types=[
            pltpu.VMEM((_S_CHUNK_ROWS, 128), jnp.int32),
            pltpu.VMEM((_S_CHUNK_ROWS * 128, 16), _f32),
            pltpu.VMEM_SHARED((_NPAD, 16), _f32),
        ],
        compiler_params=pltpu.CompilerParams(use_tc_tiling_on_sc=False),
    )
    return gather(_sc_gather_body), scatter(_sc_scatter_body)


def _sc_gather_body(tabS, tabN, seg2d, nbr2d, xs_out, xn_out,
                    segv, nbrv, bufS, bufN, semS0, semN0, semS1, semN1):
    wid = lax.axis_index("s") * 2 + lax.axis_index("c")
    sems = ((semS0, semN0), (semS1, semN1))

    def fire(k, par):
        semS, semN = sems[par]
        row0 = wid * _ROWS_PER_W + k * _G_CHUNK_ROWS
        pltpu.sync_copy(seg2d.at[pl.ds(row0, _G_CHUNK_ROWS)], segv.at[par])
        pltpu.sync_copy(nbr2d.at[pl.ds(row0, _G_CHUNK_ROWS)], nbrv.at[par])

        def sub(j, c2):
            pltpu.async_copy(tabS.at[segv.at[par].at[j]],
                             bufS.at[par].at[pl.ds(j * 128, 128)], semS)
            pltpu.async_copy(tabN.at[nbrv.at[par].at[j]],
                             bufN.at[par].at[pl.ds(j * 128, 128)], semN)
            return c2

        lax.fori_loop(0, _G_CHUNK_ROWS, sub, 0)

    def drain_write(k, par):
        semS, semN = sems[par]
        pltpu.make_async_copy(tabS.at[pl.ds(0, _G_CHUNK_ROWS * 128)],
                              bufS.at[par], semS).wait()
        pltpu.make_async_copy(tabN.at[pl.ds(0, _G_CHUNK_ROWS * 128)],
                              bufN.at[par], semN).wait()
        e0 = (wid * _ROWS_PER_W + k * _G_CHUNK_ROWS) * 128
        pltpu.sync_copy(bufS.at[par], xs_out.at[pl.ds(e0, _G_CHUNK_ROWS * 128)])
        pltpu.sync_copy(bufN.at[par], xn_out.at[pl.ds(e0, _G_CHUNK_ROWS * 128)])

    fire(0, 0)

    def pair(i, carry):
        k0 = 2 * i

        fire(k0 + 1, 1)
        drain_write(k0, 0)

        @pl.when(k0 + 2 < _G_CHUNKS)
        def _():
            fire(k0 + 2, 0)

        drain_write(k0 + 1, 1)
        return carry

    lax.fori_loop(0, _G_CHUNKS // 2, pair, 0)


def _sc_scatter_body(eff, seg2d, zeros_tab, out, segv, valv, shared):
    cid = lax.axis_index("c")
    sid = lax.axis_index("s")
    wid = sid * 2 + cid

    pltpu.sync_copy(zeros_tab.at[pl.ds(sid * _NSLICE, _NSLICE)],
                    shared.at[pl.ds(sid * _NSLICE, _NSLICE)])
    plsc.subcore_barrier()

    def chunk(k, carry):
        row0 = wid * _ROWS_PER_W + k * _S_CHUNK_ROWS
        pltpu.sync_copy(seg2d.at[pl.ds(row0, _S_CHUNK_ROWS)], segv)
        pltpu.sync_copy(eff.at[pl.ds(row0 * 128, _S_CHUNK_ROWS * 128)], valv)

        def sub(j, c2):
            pltpu.sync_copy(valv.at[pl.ds(j * 128, 128)],
                            shared.at[segv.at[j]], add=True)
            return c2

        lax.fori_loop(0, _S_CHUNK_ROWS, sub, 0)
        return carry

    lax.fori_loop(0, _S_CHUNKS, chunk, 0)
    plsc.subcore_barrier()
    pltpu.sync_copy(shared.at[pl.ds(sid * _NSLICE, _NSLICE)],
                    out.at[pl.ds(cid * _NPAD + sid * _NSLICE, _NSLICE)])


def kernel(grid_obs, edge_index, params):
    seg = edge_index[0]
    nbr = edge_index[1]
    pad = jnp.zeros((_EP - _E,), jnp.int32)
    seg_pad = jnp.concatenate([seg, pad])
    seg2d = seg_pad.reshape(_IDX_ROWS, 128)
    nbr2d = jnp.concatenate([nbr, pad]).reshape(_IDX_ROWS, 128)

    s = jnp.arange(_EP, dtype=jnp.int32)
    blk, rem = s // _EBLK, s % _EBLK
    pvec = blk * _EBLK + 4 * (rem // 8) + 2048 * ((rem % 8) // 4) + rem % 4
    seg2d_scat = seg_pad[pvec].reshape(_IDX_ROWS, 128)

    zeros_hid = jnp.zeros((_N, 4), _f32)
    cells = jnp.concatenate(
        [grid_obs[0], zeros_hid, grid_obs[1], zeros_hid], axis=1)
    zeros_tab = jnp.zeros((_NPAD, 16), _f32)

    wcet = _wlist(params, "cet")
    wnet = _wlist(params, "net")
    wcat = _wlist(params, "cat")
    weat = _wlist(params, "eat")
    wapp = _wlist(params, "app")
    weffd = _edge_weights(params)

    tabS, tabN = _call_init_tables(cells, wcet, wnet)
    sc_gather, sc_scatter = _sc_kernels()

    preds = []
    for _ in range(2):
        xs, xn = sc_gather(tabS, tabN, seg2d, nbr2d)
        eff8 = _call_edge(xs.reshape(_EP // 4, 128), xn.reshape(_EP // 4, 128),
                          weffd)
        partials = sc_scatter(eff8.reshape(_EP, 16), seg2d_scat, zeros_tab)
        pred, cells, tabS, tabN = _call_update(
            cells, partials[:_N], partials[_NPAD:_NPAD + _N],
            wcat, weat, wapp, wcet, wnet)
        preds.append(pred.reshape(_N, 2, 4).transpose(1, 0, 2))

    return jnp.stack(preds, axis=1)
